# baseline (speedup 1.0000x reference)
import functools

import jax
import jax.numpy as jnp
from jax import lax
from jax.experimental import pallas as pl
from jax.experimental.pallas import tpu as pltpu
from jax.experimental.pallas import tpu_sc as plsc

VOCAB = 32128
D = 768
N_ERR = 5
B, S = 4, 512
N_TOK = B * S
NC, NS = 2, 16
NW = NC * NS
TPW = N_TOK // NW

VB = 2048
NVB = -(-VOCAB // VB)
RB = N_TOK // 2
EPAD = 8


_SC_MESH = plsc.VectorSubcoreMesh(core_axis_name="c", subcore_axis_name="s")


@functools.partial(
    pl.kernel,
    out_type=jax.ShapeDtypeStruct((N_TOK, D), jnp.float32),
    mesh=_SC_MESH,
    scratch_types=[
        pltpu.VMEM((TPW,), jnp.int32),
        pltpu.VMEM((TPW, D), jnp.float32),
        pltpu.SemaphoreType.DMA,
    ],
)
def _gather_sc(tok_hbm, emb_hbm, out_hbm, idx_v, rows_v, sem):
    w = lax.axis_index("s") * NC + lax.axis_index("c")
    base = w * TPW
    pltpu.sync_copy(tok_hbm.at[pl.ds(base, TPW)], idx_v)
    pltpu.async_copy(emb_hbm.at[idx_v], rows_v, sem).wait()
    pltpu.sync_copy(rows_v, out_hbm.at[pl.ds(base, TPW)])


def _heads_tc(gat_ref, seg_ref, wc_ref, bc_ref, we_ref, be_ref,
              oc_ref, oe_ref, mbf_ref, wbf_ref):
    v = pl.program_id(0)
    r = pl.program_id(1)

    @pl.when((v == 0) & (r == 0))
    def _():
        segs = seg_ref[...]
        m_iota = lax.broadcasted_iota(jnp.int32, (S, S), 1)
        we_bf = we_ref[...].astype(jnp.bfloat16)
        for b in range(B):
            oh = (segs[:, b:b + 1] == m_iota).astype(jnp.bfloat16)
            g_b = gat_ref[pl.ds(b * S, S), :].astype(jnp.bfloat16)
            m_b = lax.dot_general(
                oh, g_b, (((0,), (0,)), ((), ())),
                preferred_element_type=jnp.float32)
            m_bf = m_b.astype(jnp.bfloat16)
            mbf_ref[pl.ds(b * S, S), :] = m_bf
            oe_ref[pl.ds(b * S, S), :] = lax.dot_general(
                m_bf, we_bf, (((1,), (1,)), ((), ())),
                preferred_element_type=jnp.float32) + be_ref[...]

    @pl.when(r == 0)
    def _():
        wbf_ref[...] = wc_ref[...].astype(jnp.bfloat16)

    oc_ref[...] = lax.dot_general(
        mbf_ref[pl.ds(r * RB, RB), :], wbf_ref[...],
        (((1,), (1,)), ((), ())),
        preferred_element_type=jnp.float32) + bc_ref[...]


_heads_call = pl.pallas_call(
    _heads_tc,
    grid=(NVB, N_TOK // RB),
    in_specs=[
        pl.BlockSpec((N_TOK, D), lambda v, r: (0, 0)),
        pl.BlockSpec((S, B), lambda v, r: (0, 0)),
        pl.BlockSpec((VB, D), lambda v, r: (v, 0)),
        pl.BlockSpec((1, VB), lambda v, r: (0, v)),
        pl.BlockSpec((EPAD, D), lambda v, r: (0, 0)),
        pl.BlockSpec((1, EPAD), lambda v, r: (0, 0)),
    ],
    out_specs=[
        pl.BlockSpec((RB, VB), lambda v, r: (r, v)),
        pl.BlockSpec((N_TOK, EPAD), lambda v, r: (0, 0)),
    ],
    out_shape=[
        jax.ShapeDtypeStruct((N_TOK, VOCAB), jnp.float32),
        jax.ShapeDtypeStruct((N_TOK, EPAD), jnp.float32),
    ],
    scratch_shapes=[pltpu.VMEM((N_TOK, D), jnp.bfloat16),
                    pltpu.VMEM((VB, D), jnp.bfloat16)],
)


def kernel(inputs, indexs, emb_table, W_err, b_err, W_cor, b_cor):
    tok = inputs.reshape(N_TOK).astype(jnp.int32)
    seg_t = indexs.astype(jnp.int32).T
    gathered = _gather_sc(tok, emb_table)

    we_pad = jnp.zeros((EPAD, D), jnp.float32).at[:N_ERR].set(W_err)
    be_pad = jnp.zeros((1, EPAD), jnp.float32).at[0, :N_ERR].set(b_err)
    oc, oe = _heads_call(gathered, seg_t, W_cor, b_cor.reshape(1, VOCAB),
                         we_pad, be_pad)
    out_err = oe[:, :N_ERR].reshape(B, S, N_ERR)
    out_cor = oc.reshape(B, S, VOCAB)
    return (out_err, out_cor)


# speedup vs baseline: 1.2434x; 1.2434x over previous
import functools

import jax
import jax.numpy as jnp
from jax import lax
from jax.experimental import pallas as pl
from jax.experimental.pallas import tpu as pltpu
from jax.experimental.pallas import tpu_sc as plsc

VOCAB = 32128
D = 768
N_ERR = 5
B, S = 4, 512
N_TOK = B * S
NC, NS = 2, 16
NW = NC * NS
TPW = N_TOK // NW

VB = 1792
NVB = -(-VOCAB // VB)
EPAD = 8


_SC_MESH = plsc.VectorSubcoreMesh(core_axis_name="c", subcore_axis_name="s")


@functools.partial(
    pl.kernel,
    out_type=jax.ShapeDtypeStruct((N_TOK, D), jnp.float32),
    mesh=_SC_MESH,
    scratch_types=[
        pltpu.VMEM((TPW // 2,), jnp.int32),
        pltpu.VMEM((TPW // 2,), jnp.int32),
        pltpu.VMEM((TPW // 2, D), jnp.float32),
        pltpu.VMEM((TPW // 2, D), jnp.float32),
        pltpu.SemaphoreType.DMA,
        pltpu.SemaphoreType.DMA,
        pltpu.SemaphoreType.DMA,
    ],
)
def _gather_sc(tok_hbm, emb_hbm, out_hbm, idx0, idx1, rows0, rows1,
               sem0, sem1, semw):
    w = lax.axis_index("s") * NC + lax.axis_index("c")
    base = w * TPW
    half = TPW // 2
    pltpu.sync_copy(tok_hbm.at[pl.ds(base, half)], idx0)
    g0 = pltpu.async_copy(emb_hbm.at[idx0], rows0, sem0)
    pltpu.sync_copy(tok_hbm.at[pl.ds(base + half, half)], idx1)
    g1 = pltpu.async_copy(emb_hbm.at[idx1], rows1, sem1)
    g0.wait()
    w0 = pltpu.async_copy(rows0, out_hbm.at[pl.ds(base, half)], semw)
    g1.wait()
    w1 = pltpu.async_copy(rows1, out_hbm.at[pl.ds(base + half, half)], semw)
    w0.wait()
    w1.wait()


def _heads_tc(gat_ref, seg_ref, wc_ref, bc_ref, we_ref, be_ref,
              oc_ref, oe_ref, mbf_ref):
    @pl.when(pl.program_id(0) == 0)
    def _():
        segs = seg_ref[...]
        m_iota = lax.broadcasted_iota(jnp.int32, (S, S), 1)
        we_bf = we_ref[...].astype(jnp.bfloat16)
        for b in range(B):
            oh = (segs[:, b:b + 1] == m_iota).astype(jnp.bfloat16)
            g_b = gat_ref[pl.ds(b * S, S), :].astype(jnp.bfloat16)
            m_b = lax.dot_general(
                oh, g_b, (((0,), (0,)), ((), ())),
                preferred_element_type=jnp.float32)
            m_bf = m_b.astype(jnp.bfloat16)
            mbf_ref[pl.ds(b * S, S), :] = m_bf
            oe_ref[pl.ds(b * S, S), :] = lax.dot_general(
                m_bf, we_bf, (((1,), (1,)), ((), ())),
                preferred_element_type=jnp.float32) + be_ref[...]

    oc_ref[...] = lax.dot_general(
        mbf_ref[...], wc_ref[...].astype(jnp.bfloat16),
        (((1,), (1,)), ((), ())),
        preferred_element_type=jnp.float32) + bc_ref[...]


_heads_call = pl.pallas_call(
    _heads_tc,
    grid=(NVB,),
    in_specs=[
        pl.BlockSpec((N_TOK, D), lambda j: (0, 0)),
        pl.BlockSpec((S, B), lambda j: (0, 0)),
        pl.BlockSpec((VB, D), lambda j: (j, 0)),
        pl.BlockSpec((1, VB), lambda j: (0, j)),
        pl.BlockSpec((EPAD, D), lambda j: (0, 0)),
        pl.BlockSpec((1, EPAD), lambda j: (0, 0)),
    ],
    out_specs=[
        pl.BlockSpec((N_TOK, VB), lambda j: (0, j)),
        pl.BlockSpec((N_TOK, EPAD), lambda j: (0, 0)),
    ],
    out_shape=[
        jax.ShapeDtypeStruct((N_TOK, VOCAB), jnp.float32),
        jax.ShapeDtypeStruct((N_TOK, EPAD), jnp.float32),
    ],
    scratch_shapes=[pltpu.VMEM((N_TOK, D), jnp.bfloat16)],
)


def kernel(inputs, indexs, emb_table, W_err, b_err, W_cor, b_cor):
    tok = inputs.reshape(N_TOK).astype(jnp.int32)
    seg_t = indexs.astype(jnp.int32).T
    gathered = _gather_sc(tok, emb_table)

    we_pad = jnp.zeros((EPAD, D), jnp.float32).at[:N_ERR].set(W_err)
    be_pad = jnp.zeros((1, EPAD), jnp.float32).at[0, :N_ERR].set(b_err)
    oc, oe = _heads_call(gathered, seg_t, W_cor, b_cor.reshape(1, VOCAB),
                         we_pad, be_pad)
    out_err = oe[:, :N_ERR].reshape(B, S, N_ERR)
    out_cor = oc.reshape(B, S, VOCAB)
    return (out_err, out_cor)
